# Initial kernel scaffold; baseline (speedup 1.0000x reference)
import jax
import jax.numpy as jnp
from jax.experimental import pallas as pl


def kernel(concept, table):
    raise NotImplementedError("write your pallas kernel here")


# speedup vs baseline: 4.0914x; 4.0914x over previous
import functools

import jax
import jax.numpy as jnp
from jax import lax
from jax.experimental import pallas as pl
from jax.experimental.pallas import tpu as pltpu
from jax.experimental.pallas import tpu_sc as plsc

NC = 2
NS = 16
NW = NC * NS
G = 128
GPB = 4
CB = G * GPB


@functools.lru_cache(maxsize=None)
def _build_lookup(B, V, D):
    R = B // NW
    NBLK = R // CB
    mesh = plsc.VectorSubcoreMesh(
        core_axis_name="c", subcore_axis_name="s",
        num_cores=NC, num_subcores=NS)

    @functools.partial(
        pl.kernel,
        out_type=jax.ShapeDtypeStruct((B, D), jnp.float32),
        mesh=mesh,
        scratch_types=[
            pltpu.VMEM((R // G, G), jnp.int32),
            pltpu.VMEM((CB, D), jnp.float32),
            pltpu.SemaphoreType.DMA,
        ],
        compiler_params=pltpu.CompilerParams(use_tc_tiling_on_sc=False),
    )
    def lookup(idx_hbm, table_hbm, out_hbm, idx_v, rows_v, sem):
        wid = lax.axis_index("s") * NC + lax.axis_index("c")
        base = wid * R
        pltpu.sync_copy(idx_hbm.at[wid], idx_v)

        @pl.loop(0, NBLK)
        def _block(g):
            handles = []
            for j in range(GPB):
                handles.append(pltpu.async_copy(
                    table_hbm.at[idx_v.at[g * GPB + j]],
                    rows_v.at[pl.ds(j * G, G)], sem))
            for h in handles:
                h.wait()
            pltpu.sync_copy(rows_v, out_hbm.at[pl.ds(base + g * CB, CB)])

    return lookup


def kernel(concept, table):
    S, T = concept.shape
    V, D = table.shape
    B = S * T
    idx = concept.reshape(NW, (B // NW) // G, G).astype(jnp.int32)
    out = _build_lookup(B, V, D)(idx, table)
    return out.reshape(S, T, D)


# speedup vs baseline: 4.2173x; 1.0308x over previous
import functools

import jax
import jax.numpy as jnp
from jax import lax
from jax.experimental import pallas as pl
from jax.experimental.pallas import tpu as pltpu
from jax.experimental.pallas import tpu_sc as plsc

NC = 2
NS = 16
NW = NC * NS
G = 128
GPB = 2
CB = G * GPB
NBUF = 4


@functools.lru_cache(maxsize=None)
def _build_lookup(B, V, D):
    R = B // NW
    NBLK = R // CB
    mesh = plsc.VectorSubcoreMesh(
        core_axis_name="c", subcore_axis_name="s",
        num_cores=NC, num_subcores=NS)

    @functools.partial(
        pl.kernel,
        out_type=jax.ShapeDtypeStruct((B, D), jnp.float32),
        mesh=mesh,
        scratch_types=[
            pltpu.VMEM((R // G, G), jnp.int32),
            [pltpu.VMEM((CB, D), jnp.float32)] * NBUF,
            [pltpu.SemaphoreType.DMA] * NBUF,
            [pltpu.SemaphoreType.DMA] * NBUF,
        ],
        compiler_params=pltpu.CompilerParams(use_tc_tiling_on_sc=False),
    )
    def lookup(idx_hbm, table_hbm, out_hbm, idx_v, bufs, gsems, wsems):
        wid = lax.axis_index("s") * NC + lax.axis_index("c")
        base = wid * R
        pltpu.sync_copy(idx_hbm.at[wid], idx_v)

        @pl.loop(0, NBLK, step=NBUF)
        def _block(t):
            gh = []
            for b in range(NBUF):
                for j in range(GPB):
                    gh.append(pltpu.async_copy(
                        table_hbm.at[idx_v.at[(t + b) * GPB + j]],
                        bufs[b].at[pl.ds(j * G, G)], gsems[b]))
            wh = []
            for b in range(NBUF):
                for j in range(GPB):
                    gh[b * GPB + j].wait()
                wh.append(pltpu.async_copy(
                    bufs[b], out_hbm.at[pl.ds(base + (t + b) * CB, CB)],
                    wsems[b]))
            for h in wh:
                h.wait()

    return lookup


def kernel(concept, table):
    S, T = concept.shape
    V, D = table.shape
    B = S * T
    idx = concept.reshape(NW, (B // NW) // G, G).astype(jnp.int32)
    out = _build_lookup(B, V, D)(idx, table)
    return out.reshape(S, T, D)
